# baseline (device time: 62466 ns/iter reference)
import os

import jax
import jax.numpy as jnp
from jax import lax
from jax.experimental import pallas as pl
from jax.experimental.pallas import tpu as pltpu

N_DEV = 4
KB = int(os.environ.get("KERNEL_KB", "512"))
NO_RDMA = os.environ.get("KERNEL_NO_RDMA") == "1"
D_ORDER = (2, 1, 3, 0)


def kernel(x, w_mat, scale_x, scale_w):
    m_per, k = x.shape
    _, n = w_mat.shape
    n_per = n // N_DEV
    m = m_per * N_DEV
    k_chunks = k // KB

    x8 = x.astype(jnp.float8_e5m2)
    w8 = w_mat.astype(jnp.float8_e5m2)

    def body(x_ref, w_ref, sx_ref, sw_ref, out_ref,
             wbf, sendb, recvb, copy_sems, send_sems, recv_sems):
        my = lax.axis_index("i")

        def dest(d):
            return lax.rem(my + d, N_DEV)

        def w_block_start(di, d):
            for kc in range(k_chunks):
                pltpu.make_async_copy(
                    w_ref.at[pl.ds(kc * KB, KB), pl.ds(dest(d) * n_per, n_per)],
                    wbf.at[di % 2, pl.ds(kc * KB, KB), :],
                    copy_sems.at[di % 2, kc],
                ).start()

        def w_block_wait(di, d):
            for kc in range(k_chunks):
                pltpu.make_async_copy(
                    w_ref.at[pl.ds(kc * KB, KB), pl.ds(dest(d) * n_per, n_per)],
                    wbf.at[di % 2, pl.ds(kc * KB, KB), :],
                    copy_sems.at[di % 2, kc],
                ).wait()

        barrier = pltpu.get_barrier_semaphore()
        for d in (1, 2, 3):
            pl.semaphore_signal(
                barrier, inc=1, device_id=(dest(d),),
                device_id_type=pl.DeviceIdType.MESH,
            )

        w_block_start(0, D_ORDER[0])
        w_block_start(1, D_ORDER[1])

        scale = sx_ref[0] * sw_ref[0]
        rdmas = []
        for di, d in enumerate(D_ORDER):
            w_block_wait(di, d)
            final = (
                jnp.dot(x_ref[...], wbf[di % 2],
                        preferred_element_type=jnp.float32)
                * scale
            )
            if di + 2 < N_DEV:
                w_block_start(di + 2, D_ORDER[di + 2])

            if d == 0:
                out_ref[pl.ds(my * m_per, m_per), :] = final
            else:
                sendb[d - 1] = final.astype(jnp.bfloat16)
                if not NO_RDMA:
                    if di == 0:
                        pl.semaphore_wait(barrier, N_DEV - 1)
                    rdma = pltpu.make_async_remote_copy(
                        src_ref=sendb.at[d - 1],
                        dst_ref=recvb.at[d - 1],
                        send_sem=send_sems.at[d - 1],
                        recv_sem=recv_sems.at[d - 1],
                        device_id=(dest(d),),
                        device_id_type=pl.DeviceIdType.MESH,
                    )
                    rdma.start()
                    rdmas.append(rdma)

        if NO_RDMA:
            pl.semaphore_wait(barrier, N_DEV - 1)

        drain = [d for d in D_ORDER if d != 0]
        for d in drain:
            if not rdmas:
                continue
            rdma = rdmas[drain.index(d)]
            src_peer = lax.rem(my - d + N_DEV, N_DEV)
            recv = pltpu.make_async_remote_copy(
                src_ref=sendb.at[d - 1],
                dst_ref=recvb.at[d - 1],
                send_sem=send_sems.at[d - 1],
                recv_sem=recv_sems.at[d - 1],
                device_id=(src_peer,),
                device_id_type=pl.DeviceIdType.MESH,
            )
            recv.wait_recv()
            out_ref[pl.ds(src_peer * m_per, m_per), :] = recvb[d - 1].astype(
                jnp.float32
            )
            rdma.wait_send()

    return pl.pallas_call(
        body,
        out_shape=jax.ShapeDtypeStruct((m, n_per), jnp.float32),
        in_specs=[
            pl.BlockSpec(memory_space=pltpu.VMEM),
            pl.BlockSpec(memory_space=pl.MemorySpace.ANY),
            pl.BlockSpec(memory_space=pltpu.SMEM),
            pl.BlockSpec(memory_space=pltpu.SMEM),
        ],
        out_specs=pl.BlockSpec(memory_space=pltpu.VMEM),
        scratch_shapes=[
            pltpu.VMEM((2, k, n_per), jnp.float8_e5m2),
            pltpu.VMEM((N_DEV - 1, m_per, n_per), jnp.bfloat16),
            pltpu.VMEM((N_DEV - 1, m_per, n_per), jnp.bfloat16),
            pltpu.SemaphoreType.DMA((2, k // KB)),
            pltpu.SemaphoreType.DMA((N_DEV - 1,)),
            pltpu.SemaphoreType.DMA((N_DEV - 1,)),
        ],
        compiler_params=pltpu.CompilerParams(
            collective_id=0,
            vmem_limit_bytes=48 * 1024 * 1024,
        ),
    )(x8, w8, scale_x, scale_w)


# device time: 45940 ns/iter; 1.3597x vs baseline; 1.3597x over previous
import os

import jax
import jax.numpy as jnp
from jax import lax
from jax.experimental import pallas as pl
from jax.experimental.pallas import tpu as pltpu

N_DEV = 4
KB = int(os.environ.get("KERNEL_KB", "512"))
NBUF = int(os.environ.get("KERNEL_NBUF", "10"))
NO_RDMA = os.environ.get("KERNEL_NO_RDMA") == "1"
D_ORDER = (2, 1, 3, 0)
F8 = jnp.float8_e5m2


def kernel(x, w_mat, scale_x, scale_w):
    m_per, k = x.shape
    _, n = w_mat.shape
    n_per = n // N_DEV
    m = m_per * N_DEV
    half = m_per // 2
    k_chunks = k // KB
    steps = [(di, d, kc) for di, d in enumerate(D_ORDER) for kc in range(k_chunks)]
    nsteps = len(steps)
    xc = m_per // 4

    def body(x_ref, w_ref, sx_ref, sw_ref, out_ref,
             x8, xstage, wbuf, w8, sendb, recvb,
             xcopy_sems, copy_sems, send_sems, recv_sems):
        my = lax.axis_index("i")

        def dest(d):
            return lax.rem(my + d, N_DEV)

        def w_copy(step, slot):
            _, d, kc = steps[step]
            return pltpu.make_async_copy(
                w_ref.at[pl.ds(kc * KB, KB), pl.ds(dest(d) * n_per, n_per)],
                wbuf.at[slot],
                copy_sems.at[slot],
            )

        def x_copy(c, slot):
            return pltpu.make_async_copy(
                x_ref.at[pl.ds(c * xc, xc), :],
                xstage.at[slot],
                xcopy_sems.at[slot],
            )

        barrier = pltpu.get_barrier_semaphore()
        for d in (1, 2, 3):
            pl.semaphore_signal(
                barrier, inc=1, device_id=(dest(d),),
                device_id_type=pl.DeviceIdType.MESH,
            )

        for s in range(min(NBUF, nsteps)):
            w_copy(s, s % NBUF).start()
        x_copy(0, 0).start()
        x_copy(1, 1).start()

        for c in (0, 1):
            x_copy(c, c % 2).wait()
            if c + 2 < 4:
                x_copy(c + 2, c % 2).start()
            x8[pl.ds(c * xc, xc), :] = xstage[c % 2].astype(F8)

        scale = sx_ref[0] * sw_ref[0]
        rdmas = []

        def send_half(bi, d, h, val):
            sendb[bi, pl.ds(h * half, half), :] = val.astype(jnp.bfloat16)
            if NO_RDMA:
                return
            rdma = pltpu.make_async_remote_copy(
                src_ref=sendb.at[bi, pl.ds(h * half, half), :],
                dst_ref=recvb.at[bi, pl.ds(h * half, half), :],
                send_sem=send_sems.at[bi, h],
                recv_sem=recv_sems.at[bi, h],
                device_id=(dest(d),),
                device_id_type=pl.DeviceIdType.MESH,
            )
            rdma.start()
            rdmas.append(rdma)

        first_send_done = False
        for di, d in enumerate(D_ORDER):
            for kc in range(k_chunks):
                step = di * k_chunks + kc
                slot = step % NBUF
                w_copy(step, slot).wait()
                w8[pl.ds(kc * KB, KB), pl.ds(dest(d) * n_per, n_per)] = (
                    wbuf[slot].astype(F8)
                )
                if step + NBUF < nsteps:
                    w_copy(step + NBUF, slot).start()

            wcols = w8[:, pl.ds(dest(d) * n_per, n_per)]
            if d == 0:
                final = (
                    jnp.dot(x8[...], wcols, preferred_element_type=jnp.float32)
                    * scale
                )
                out_ref[pl.ds(my * m_per, m_per), :] = final
                continue

            bi = d - 1
            f0 = (
                jnp.dot(x8[pl.ds(0, half), :], wcols,
                        preferred_element_type=jnp.float32)
                * scale
            )
            if not first_send_done and not NO_RDMA:
                pl.semaphore_wait(barrier, N_DEV - 1)
                first_send_done = True
            send_half(bi, d, 0, f0)
            if di == 0:
                for c in (2, 3):
                    x_copy(c, c % 2).wait()
                    x8[pl.ds(c * xc, xc), :] = xstage[c % 2].astype(F8)
            f1 = (
                jnp.dot(x8[pl.ds(half, half), :], wcols,
                        preferred_element_type=jnp.float32)
                * scale
            )
            send_half(bi, d, 1, f1)

        if NO_RDMA:
            pl.semaphore_wait(barrier, N_DEV - 1)
            rdmas.clear()

        k_i = 0
        for d in (dd for dd in D_ORDER if dd != 0):
            bi = d - 1
            src_peer = lax.rem(my - d + N_DEV, N_DEV)
            for h in (0, 1):
                if not rdmas:
                    continue
                recv = pltpu.make_async_remote_copy(
                    src_ref=sendb.at[bi, pl.ds(h * half, half), :],
                    dst_ref=recvb.at[bi, pl.ds(h * half, half), :],
                    send_sem=send_sems.at[bi, h],
                    recv_sem=recv_sems.at[bi, h],
                    device_id=(src_peer,),
                    device_id_type=pl.DeviceIdType.MESH,
                )
                recv.wait_recv()
                out_ref[pl.ds(src_peer * m_per + h * half, half), :] = (
                    recvb[bi, pl.ds(h * half, half), :].astype(jnp.float32)
                )
                rdmas[k_i].wait_send()
                k_i += 1

    return pl.pallas_call(
        body,
        out_shape=jax.ShapeDtypeStruct((m, n_per), jnp.float32),
        in_specs=[
            pl.BlockSpec(memory_space=pl.MemorySpace.ANY),
            pl.BlockSpec(memory_space=pl.MemorySpace.ANY),
            pl.BlockSpec(memory_space=pltpu.SMEM),
            pl.BlockSpec(memory_space=pltpu.SMEM),
        ],
        out_specs=pl.BlockSpec(memory_space=pltpu.VMEM),
        scratch_shapes=[
            pltpu.VMEM((m_per, k), F8),
            pltpu.VMEM((2, m_per // 4, k), jnp.float32),
            pltpu.VMEM((NBUF, KB, n_per), jnp.float32),
            pltpu.VMEM((k, n), F8),
            pltpu.VMEM((N_DEV - 1, m_per, n_per), jnp.bfloat16),
            pltpu.VMEM((N_DEV - 1, m_per, n_per), jnp.bfloat16),
            pltpu.SemaphoreType.DMA((2,)),
            pltpu.SemaphoreType.DMA((NBUF,)),
            pltpu.SemaphoreType.DMA((N_DEV - 1, 2)),
            pltpu.SemaphoreType.DMA((N_DEV - 1, 2)),
        ],
        compiler_params=pltpu.CompilerParams(
            collective_id=0,
            vmem_limit_bytes=56 * 1024 * 1024,
        ),
    )(x, w_mat, scale_x, scale_w)
